# baseline (speedup 1.0000x reference)
import functools

import jax
import jax.numpy as jnp
from jax import lax
from jax.experimental import pallas as pl
from jax.experimental.pallas import tpu as pltpu
from jax.experimental.pallas import tpu_sc as plsc

N_ASSETS = 10
SUB_SIZE = 100000
TOTAL_VOCAB = N_ASSETS * SUB_SIZE
EMBED_DIM = 32
BATCH = 16384

_INFO = plsc.get_sparse_core_info()
_NC = _INFO.num_cores
_NS = _INFO.num_subcores
_LANES = _INFO.num_lanes
_NW = _NC * _NS
_BPW = BATCH // _NW
_CHUNK = 128
_NCHUNK = _BPW // _CHUNK


def _sc_body(asset_hbm, shape_hbm, table_hbm, out_hbm,
             asset_v, shape_v, idx_v, rows_v, sem):
    wid = lax.axis_index("s") * _NC + lax.axis_index("c")
    base = wid * _BPW

    pltpu.sync_copy(asset_hbm.at[pl.ds(base, _BPW)], asset_v)
    pltpu.sync_copy(shape_hbm.at[pl.ds(base, _BPW)], shape_v)

    copies = []
    for c in range(_NCHUNK):
        for i in range(_CHUNK // _LANES):
            off = c * _CHUNK + i * _LANES
            a = asset_v[pl.ds(off, _LANES)]
            s = shape_v[pl.ds(off, _LANES)]
            idx_v[pl.ds(off, _LANES)] = a * SUB_SIZE + s
        cp = pltpu.make_async_copy(
            table_hbm.at[idx_v.at[pl.ds(c * _CHUNK, _CHUNK)]],
            rows_v.at[pl.ds(c * _CHUNK, _CHUNK)],
            sem,
        )
        cp.start()
        copies.append(cp)

    for cp in copies:
        cp.wait()

    pltpu.sync_copy(rows_v, out_hbm.at[pl.ds(base, _BPW)])


@jax.jit
def _lookup(asset_index, shape_index, table):
    mesh = plsc.VectorSubcoreMesh(core_axis_name="c", subcore_axis_name="s")
    fn = pl.kernel(
        _sc_body,
        out_type=jax.ShapeDtypeStruct((BATCH, EMBED_DIM), jnp.float32),
        mesh=mesh,
        scratch_types=[
            pltpu.VMEM((_BPW,), jnp.int32),
            pltpu.VMEM((_BPW,), jnp.int32),
            pltpu.VMEM((_BPW,), jnp.int32),
            pltpu.VMEM((_BPW, EMBED_DIM), jnp.float32),
            pltpu.SemaphoreType.DMA,
        ],
        compiler_params=pltpu.CompilerParams(use_tc_tiling_on_sc=False),
    )
    return fn(asset_index, shape_index, table)


def kernel(asset_index, shape_index, table):
    return _lookup(asset_index.astype(jnp.int32),
                   shape_index.astype(jnp.int32),
                   table)
